# Initial kernel scaffold; baseline (speedup 1.0000x reference)
import jax
import jax.numpy as jnp
from jax.experimental import pallas as pl


def kernel(edge_params, velocities):
    raise NotImplementedError("write your pallas kernel here")


# speedup vs baseline: 70.5286x; 70.5286x over previous
import jax
import jax.numpy as jnp
from jax import lax
from jax.experimental import pallas as pl
from jax.experimental.pallas import tpu as pltpu
from jax.experimental.pallas import tpu_sc as plsc

H = 4096
W = 4096
L = 64
N = 100000

NC = 2
NS = 16
NW = NC * NS
ROWS_PER_TILE = H // NW
SB = 16
SB_ROWS = ROWS_PER_TILE // SB
SB_WORDS = SB_ROWS * W

NSEG = 8
SEGE = 12544
NPAD = NSEG * SEGE
CPS = SEGE // 16
NCHUNK = NPAD // 16

CAPL = 12288
NB = 10


def _sc_body(x0_hbm, y_hbm, out_hbm,
             x0_vm, y_vm, lbuf, cbuf, rbuf0, rbuf1, ssem):
    c = lax.axis_index("c")
    s = lax.axis_index("s")
    wid = c * NS + s
    row0 = wid * ROWS_PER_TILE
    base_word = row0 * W
    ones16 = jnp.ones((16,), jnp.float32)
    zeros16 = jnp.zeros((16,), jnp.float32)
    neg16 = jnp.full((16,), -1, jnp.int32)
    rbufs = (rbuf0, rbuf1)

    def store_desc(buf, sb):
        return pltpu.make_async_copy(
            buf, out_hbm.at[pl.ds(base_word + sb * SB_WORDS, SB_WORDS)], ssem)

    def run_batch(first, cur):
        pf = (jnp.int32(0), jnp.int32(NCHUNK))
        for seg in range(NSEG):
            def stage():
                pltpu.sync_copy(
                    x0_hbm.at[pl.ds(seg * SEGE, SEGE)], x0_vm)
                pltpu.sync_copy(
                    y_hbm.at[pl.ds(seg * SEGE, SEGE)], y_vm)
            if first:
                stage()
                lo = seg * CPS
            else:
                pl.when((seg + 1) * CPS > cur)(stage)
                lo = jnp.maximum(cur, seg * CPS)

            @pl.loop(lo, (seg + 1) * CPS, init_carry=pf)
            def seg_pf(k, pf):
                p, fs = pf
                lk = k - seg * CPS
                x0v = x0_vm[pl.ds(lk * 16, 16)]
                yv = y_vm[pl.ds(lk * 16, 16)]
                mv = (yv >= row0) & (yv < row0 + ROWS_PER_TILE)
                cnt = jnp.sum(mv.astype(jnp.int32), axis=0)
                offv = yv * W + x0v
                fits = (p + cnt) <= CAPL

                @pl.when(fits)
                def _():
                    plsc.store_compressed(
                        lbuf.at[pl.ds(p, 16)], offv, mask=mv)

                p = jnp.where(fits, p + cnt, p)
                fs = jnp.where((~fits) & (fs == NCHUNK), k, fs)
                return (p, fs)

            pf = seg_pf

        p, first_skip = pf
        lbuf[pl.ds(p, 16)] = neg16
        nlc = (p + 15) // 16

        def subband(sb):
            buf = rbufs[sb % 2]
            if sb >= 2:
                store_desc(buf, sb - 2).wait()

            if first:
                @pl.loop(0, SB_WORDS // 16, unroll=8)
                def _(i):
                    buf[pl.ds(i * 16, 16)] = zeros16
            else:
                pltpu.sync_copy(
                    out_hbm.at[pl.ds(base_word + sb * SB_WORDS, SB_WORDS)],
                    buf)

            base = base_word + sb * SB_WORDS

            @pl.loop(0, nlc)
            def _(lc):
                offv = lbuf[pl.ds(lc * 16, 16)]
                m2 = (offv >= base) & (offv < base + SB_WORDS)
                cnt2 = jnp.sum(m2.astype(jnp.int32), axis=0)
                plsc.store_compressed(cbuf.at[pl.ds(0, 16)], offv, mask=m2)

                @pl.loop(0, cnt2)
                def _(i):
                    local = cbuf[pl.ds(i, 16)][0] - base
                    for j in range(4):
                        buf[pl.ds(local + j * 16, 16)] = ones16

            store_desc(buf, sb).start()

        for sb in range(SB):
            if first:
                subband(sb)
            else:
                pl.when(p > 0)(lambda sb=sb: subband(sb))

        def drain():
            store_desc(rbufs[SB % 2], SB - 2).wait()
            store_desc(rbufs[(SB + 1) % 2], SB - 1).wait()
        if first:
            drain()
        else:
            pl.when(p > 0)(drain)

        return first_skip

    cur = run_batch(True, jnp.int32(0))

    @pl.loop(1, NB, init_carry=cur)
    def final_cur(b, cur):
        return run_batch(False, cur)


@jax.jit
def kernel(edge_params, velocities):
    ep = edge_params.astype(jnp.int32)
    x0 = ep[:, 0, 0]
    y = ep[:, 0, 1]
    pad = NPAD - N
    x0 = jnp.pad(x0, (0, pad))
    y = jnp.pad(y, (0, pad), constant_values=-1)

    mesh = plsc.VectorSubcoreMesh(
        core_axis_name="c", subcore_axis_name="s",
        num_cores=NC, num_subcores=NS)
    out = pl.kernel(
        _sc_body,
        out_type=jax.ShapeDtypeStruct((H * W,), jnp.float32),
        mesh=mesh,
        compiler_params=pltpu.CompilerParams(needs_layout_passes=False),
        scratch_types=[
            pltpu.VMEM((SEGE,), jnp.int32),
            pltpu.VMEM((SEGE,), jnp.int32),
            pltpu.VMEM((CAPL + 16,), jnp.int32),
            pltpu.VMEM((32,), jnp.int32),
            pltpu.VMEM((SB_WORDS,), jnp.float32),
            pltpu.VMEM((SB_WORDS,), jnp.float32),
            pltpu.SemaphoreType.DMA,
        ],
    )(x0, y)
    return out.reshape(H, W)


# speedup vs baseline: 77.4727x; 1.0985x over previous
import jax
import jax.numpy as jnp
from jax import lax
from jax.experimental import pallas as pl
from jax.experimental.pallas import tpu as pltpu
from jax.experimental.pallas import tpu_sc as plsc

H = 4096
W = 4096
L = 64
N = 100000

NC = 2
NS = 16
NW = NC * NS
ROWS_PER_TILE = H // NW
SB = 16
SB_ROWS = ROWS_PER_TILE // SB
SB_WORDS = SB_ROWS * W

NSEG = 8
SEGE = 12544
NPAD = NSEG * SEGE
CPS = SEGE // 16
NCHUNK = NPAD // 16

CAPL = 12288
NB = 10


def _sc_body(x0_hbm, y_hbm, out_hbm,
             x0_vm, y_vm, lbuf, cbuf, rbuf0, rbuf1, ssem, gsem):
    c = lax.axis_index("c")
    s = lax.axis_index("s")
    wid = c * NS + s
    row0 = wid * ROWS_PER_TILE
    base_word = row0 * W
    ones16 = jnp.ones((16,), jnp.float32)
    zeros16 = jnp.zeros((16,), jnp.float32)
    neg16 = jnp.full((16,), -1, jnp.int32)
    rbufs = (rbuf0, rbuf1)

    def store_desc(buf, sb):
        return pltpu.make_async_copy(
            buf, out_hbm.at[pl.ds(base_word + sb * SB_WORDS, SB_WORDS)], ssem)

    def run_batch(first, cur):
        pf = (jnp.int32(0), jnp.int32(NCHUNK))
        for seg in range(NSEG):
            def stage():
                a = pltpu.make_async_copy(
                    x0_hbm.at[pl.ds(seg * SEGE, SEGE)], x0_vm, gsem)
                b = pltpu.make_async_copy(
                    y_hbm.at[pl.ds(seg * SEGE, SEGE)], y_vm, gsem)
                a.start()
                b.start()
                a.wait()
                b.wait()
            if first:
                stage()
                lo = seg * CPS
            else:
                pl.when((seg + 1) * CPS > cur)(stage)
                lo = jnp.maximum(cur, seg * CPS)

            @pl.loop(lo, (seg + 1) * CPS, init_carry=pf,
                     unroll=2 if first else None)
            def seg_pf(k, pf):
                p, fs = pf
                lk = k - seg * CPS
                x0v = x0_vm[pl.ds(lk * 16, 16)]
                yv = y_vm[pl.ds(lk * 16, 16)]
                mv = (yv >= row0) & (yv < row0 + ROWS_PER_TILE)
                cnt = plsc.all_reduce_population_count(mv)[0]
                offv = yv * W + x0v
                fits = (p + cnt) <= CAPL

                @pl.when(fits)
                def _():
                    plsc.store_compressed(
                        lbuf.at[pl.ds(p, 16)], offv, mask=mv)

                p = jnp.where(fits, p + cnt, p)
                fs = jnp.where((~fits) & (fs == NCHUNK), k, fs)
                return (p, fs)

            pf = seg_pf

        p, first_skip = pf
        lbuf[pl.ds(p, 16)] = neg16
        nlc = (p + 15) // 16

        def subband(sb):
            buf = rbufs[sb % 2]
            if sb >= 2:
                store_desc(buf, sb - 2).wait()

            if first:
                @pl.loop(0, SB_WORDS // 16, unroll=8)
                def _(i):
                    buf[pl.ds(i * 16, 16)] = zeros16
            else:
                pltpu.sync_copy(
                    out_hbm.at[pl.ds(base_word + sb * SB_WORDS, SB_WORDS)],
                    buf)

            base = base_word + sb * SB_WORDS

            @pl.loop(0, nlc)
            def _(lc):
                offv = lbuf[pl.ds(lc * 16, 16)]
                m2 = (offv >= base) & (offv < base + SB_WORDS)
                cnt2 = plsc.all_reduce_population_count(m2)[0]
                plsc.store_compressed(cbuf.at[pl.ds(0, 16)], offv, mask=m2)

                @pl.loop(0, cnt2)
                def _(i):
                    local = cbuf[pl.ds(i, 16)][0] - base
                    for j in range(4):
                        buf[pl.ds(local + j * 16, 16)] = ones16

            store_desc(buf, sb).start()

        for sb in range(SB):
            if first:
                subband(sb)
            else:
                pl.when(p > 0)(lambda sb=sb: subband(sb))

        def drain():
            store_desc(rbufs[SB % 2], SB - 2).wait()
            store_desc(rbufs[(SB + 1) % 2], SB - 1).wait()
        if first:
            drain()
        else:
            pl.when(p > 0)(drain)

        return first_skip

    cur = run_batch(True, jnp.int32(0))

    @pl.loop(1, NB, init_carry=cur)
    def final_cur(b, cur):
        return run_batch(False, cur)


@jax.jit
def kernel(edge_params, velocities):
    ep = edge_params.astype(jnp.int32)
    x0 = ep[:, 0, 0]
    y = ep[:, 0, 1]
    pad = NPAD - N
    x0 = jnp.pad(x0, (0, pad))
    y = jnp.pad(y, (0, pad), constant_values=-1)

    mesh = plsc.VectorSubcoreMesh(
        core_axis_name="c", subcore_axis_name="s",
        num_cores=NC, num_subcores=NS)
    out = pl.kernel(
        _sc_body,
        out_type=jax.ShapeDtypeStruct((H * W,), jnp.float32),
        mesh=mesh,
        compiler_params=pltpu.CompilerParams(needs_layout_passes=False),
        scratch_types=[
            pltpu.VMEM((SEGE,), jnp.int32),
            pltpu.VMEM((SEGE,), jnp.int32),
            pltpu.VMEM((CAPL + 16,), jnp.int32),
            pltpu.VMEM((32,), jnp.int32),
            pltpu.VMEM((SB_WORDS,), jnp.float32),
            pltpu.VMEM((SB_WORDS,), jnp.float32),
            pltpu.SemaphoreType.DMA,
            pltpu.SemaphoreType.DMA,
        ],
    )(x0, y)
    return out.reshape(H, W)


# speedup vs baseline: 95.0515x; 1.2269x over previous
import jax
import jax.numpy as jnp
from jax import lax
from jax.experimental import pallas as pl
from jax.experimental.pallas import tpu as pltpu
from jax.experimental.pallas import tpu_sc as plsc

H = 4096
W = 4096
L = 64
N = 100000

NC = 2
NS = 16
NW = NC * NS
ROWS_PER_TILE = H // NW
SB = 16
SB_ROWS = ROWS_PER_TILE // SB
SB_WORDS = SB_ROWS * W

NSEG = 8
SEGE = 12544
NPAD = NSEG * SEGE
CPS = SEGE // 16
NCHUNK = NPAD // 16

CAPL = 12288
NB = 10


ZVW = 8192


def _sc_body(x0_hbm, y_hbm, out_hbm,
             x0_vm, y_vm, lbuf, ebuf, rbuf0, rbuf1, zvm, zshared,
             ssem, gsem, zsem):
    c = lax.axis_index("c")
    s = lax.axis_index("s")
    wid = c * NS + s
    row0 = wid * ROWS_PER_TILE
    base_word = row0 * W
    ones16 = jnp.ones((16,), jnp.float32)
    zeros16 = jnp.zeros((16,), jnp.float32)
    neg16 = jnp.full((16,), -1, jnp.int32)
    iota16 = lax.broadcasted_iota(jnp.int32, (16,), 0)
    rbufs = (rbuf0, rbuf1)

    @pl.loop(0, ZVW // 16, unroll=8)
    def _(i):
        zvm[pl.ds(i * 16, 16)] = zeros16

    @pl.when(s == 0)
    def _():
        for z in range(SB_WORDS // ZVW):
            pltpu.sync_copy(zvm, zshared.at[pl.ds(z * ZVW, ZVW)])
    plsc.subcore_barrier()

    def store_desc(buf, sb):
        return pltpu.make_async_copy(
            buf, out_hbm.at[pl.ds(base_word + sb * SB_WORDS, SB_WORDS)], ssem)

    def run_batch(first, cur):
        pf = (jnp.int32(0), jnp.int32(NCHUNK))
        for seg in range(NSEG):
            def stage():
                a = pltpu.make_async_copy(
                    x0_hbm.at[pl.ds(seg * SEGE, SEGE)], x0_vm, gsem)
                b = pltpu.make_async_copy(
                    y_hbm.at[pl.ds(seg * SEGE, SEGE)], y_vm, gsem)
                a.start()
                b.start()
                a.wait()
                b.wait()
            if first:
                stage()
                lo = seg * CPS
            else:
                pl.when((seg + 1) * CPS > cur)(stage)
                lo = jnp.maximum(cur, seg * CPS)

            @pl.loop(lo, (seg + 1) * CPS, init_carry=pf,
                     unroll=2 if first else None)
            def seg_pf(k, pf):
                p, fs = pf
                lk = k - seg * CPS
                x0v = x0_vm[pl.ds(lk * 16, 16)]
                yv = y_vm[pl.ds(lk * 16, 16)]
                mv = (yv - row0).astype(jnp.uint32) < ROWS_PER_TILE
                cnt = plsc.all_reduce_population_count(mv)[0]
                offv = yv * W + x0v
                fits = (p + cnt) <= CAPL

                @pl.when(fits)
                def _():
                    plsc.store_compressed(
                        lbuf.at[pl.ds(p, 16)], offv, mask=mv)

                p = jnp.where(fits, p + cnt, p)
                fs = jnp.where((~fits) & (fs == NCHUNK), k, fs)
                return (p, fs)

            pf = seg_pf

        p, first_skip = pf
        lbuf[pl.ds(p, 16)] = neg16
        nlc = (p + 15) // 16

        def subband(sb):
            buf = rbufs[sb % 2]
            if sb >= 2:
                store_desc(buf, sb - 2).wait()

            base = base_word + sb * SB_WORDS
            if first:
                pltpu.make_async_copy(zshared, buf, zsem).start()
            else:
                pltpu.sync_copy(out_hbm.at[pl.ds(base, SB_WORDS)], buf)

            @pl.loop(0, nlc, init_carry=jnp.int32(0))
            def q(lc, q):
                localv = lbuf[pl.ds(lc * 16, 16)] - base
                m2 = localv.astype(jnp.uint32) < SB_WORDS
                cnt2 = plsc.all_reduce_population_count(m2)[0]

                @pl.when(cnt2 > 0)
                def _():
                    plsc.store_compressed(
                        ebuf.at[pl.ds(q, 16)], localv, mask=m2)
                return q + cnt2

            if first:
                pltpu.make_async_copy(zshared, buf, zsem).wait()

            @pl.loop(0, (q + 15) // 16)
            def _(g):
                localv = ebuf[pl.ds(g * 16, 16)]
                mt = iota16 < jnp.minimum(q - g * 16, 16)
                idxv = localv
                for j in range(L):
                    if j:
                        idxv = idxv + 1
                    plsc.store_scatter(buf, [idxv], ones16, mask=mt)

            store_desc(buf, sb).start()

        for sb in range(SB):
            if first:
                subband(sb)
            else:
                pl.when(p > 0)(lambda sb=sb: subband(sb))

        def drain():
            store_desc(rbufs[SB % 2], SB - 2).wait()
            store_desc(rbufs[(SB + 1) % 2], SB - 1).wait()
        if first:
            drain()
        else:
            pl.when(p > 0)(drain)

        return first_skip

    cur = run_batch(True, jnp.int32(0))

    @pl.loop(1, NB, init_carry=cur)
    def final_cur(b, cur):
        return run_batch(False, cur)


@jax.jit
def kernel(edge_params, velocities):
    ep = edge_params.astype(jnp.int32)
    x0 = ep[:, 0, 0]
    y = ep[:, 0, 1]
    pad = NPAD - N
    x0 = jnp.pad(x0, (0, pad))
    y = jnp.pad(y, (0, pad), constant_values=-1)

    mesh = plsc.VectorSubcoreMesh(
        core_axis_name="c", subcore_axis_name="s",
        num_cores=NC, num_subcores=NS)
    out = pl.kernel(
        _sc_body,
        out_type=jax.ShapeDtypeStruct((H * W,), jnp.float32),
        mesh=mesh,
        compiler_params=pltpu.CompilerParams(needs_layout_passes=False),
        scratch_types=[
            pltpu.VMEM((SEGE,), jnp.int32),
            pltpu.VMEM((SEGE,), jnp.int32),
            pltpu.VMEM((CAPL + 16,), jnp.int32),
            pltpu.VMEM((CAPL + 16,), jnp.int32),
            pltpu.VMEM((SB_WORDS,), jnp.float32),
            pltpu.VMEM((SB_WORDS,), jnp.float32),
            pltpu.VMEM((ZVW,), jnp.float32),
            pltpu.VMEM_SHARED((SB_WORDS,), jnp.float32),
            pltpu.SemaphoreType.DMA,
            pltpu.SemaphoreType.DMA,
            pltpu.SemaphoreType.DMA,
        ],
    )(x0, y)
    return out.reshape(H, W)


# speedup vs baseline: 101.6148x; 1.0690x over previous
import jax
import jax.numpy as jnp
from jax import lax
from jax.experimental import pallas as pl
from jax.experimental.pallas import tpu as pltpu
from jax.experimental.pallas import tpu_sc as plsc

H = 4096
W = 4096
L = 64
N = 100000

NC = 2
NS = 16
NW = NC * NS
ROWS_PER_TILE = H // NW
SB = 16
SB_ROWS = ROWS_PER_TILE // SB
SB_WORDS = SB_ROWS * W

SEGE = 7168
NSEG = 14
NPAD = NSEG * SEGE
CPS = SEGE // 16
NCHUNK = NPAD // 16

CAPL = 12288
NB = 10
ZVW = 4096


def _sc_body(xy_hbm, out_hbm,
             xy0, xy1, lbuf, ebuf, rbuf0, rbuf1, zvm, zshared,
             ssem, gsem, zsem):
    c = lax.axis_index("c")
    s = lax.axis_index("s")
    wid = c * NS + s
    row0 = wid * ROWS_PER_TILE
    base_word = row0 * W
    ones16 = jnp.ones((16,), jnp.float32)
    zeros16 = jnp.zeros((16,), jnp.float32)
    neg16 = jnp.full((16,), -1, jnp.int32)
    iota16 = lax.broadcasted_iota(jnp.int32, (16,), 0)

    @pl.loop(0, ZVW // 16, unroll=8)
    def _(i):
        zvm[pl.ds(i * 16, 16)] = zeros16

    @pl.when(s == 0)
    def _():
        for z in range(SB_WORDS // ZVW):
            pltpu.sync_copy(zvm, zshared.at[pl.ds(z * ZVW, ZVW)])
    plsc.subcore_barrier()

    def store_desc(buf, sbd):
        return pltpu.make_async_copy(
            buf, out_hbm.at[pl.ds(base_word + sbd * SB_WORDS, SB_WORDS)],
            ssem)

    def stage_desc(seg, buf):
        return pltpu.make_async_copy(
            xy_hbm.at[pl.ds(seg * 2 * SEGE, 2 * SEGE)], buf, gsem)

    def scan_seg(segd, buf, pf, unroll):
        lol = 0 if unroll else jnp.clip(pf[2] - segd * CPS, 0, CPS)

        @pl.loop(lol, CPS, init_carry=pf, unroll=unroll)
        def out_pf(lk, pf):
            p, fs, cur = pf
            yv = buf[pl.ds(SEGE + lk * 16, 16)]
            mv = (yv - row0).astype(jnp.uint32) < ROWS_PER_TILE
            cnt = plsc.all_reduce_population_count(mv)[0]
            fits = (p + cnt) <= CAPL

            @pl.when((cnt > 0) & fits)
            def _():
                x0v = buf[pl.ds(lk * 16, 16)]
                plsc.store_compressed(
                    lbuf.at[pl.ds(p, 16)], yv * W + x0v, mask=mv)

            p = jnp.where(fits, p + cnt, p)
            fs = jnp.where(
                (~fits) & (fs == NCHUNK), segd * CPS + lk, fs)
            return (p, fs, cur)

        return out_pf

    def build(nlc, sbd, buf, first, wait_prev):
        if wait_prev is not None:
            pl.when(wait_prev)(lambda: store_desc(buf, 0).wait())
        base = base_word + sbd * SB_WORDS
        if first:
            pltpu.make_async_copy(zshared, buf, zsem).start()
        else:
            pltpu.sync_copy(out_hbm.at[pl.ds(base, SB_WORDS)], buf)

        @pl.loop(0, nlc, init_carry=jnp.int32(0))
        def q(lc, q):
            localv = lbuf[pl.ds(lc * 16, 16)] - base
            m2 = localv.astype(jnp.uint32) < SB_WORDS
            cnt2 = plsc.all_reduce_population_count(m2)[0]

            @pl.when(cnt2 > 0)
            def _():
                plsc.store_compressed(ebuf.at[pl.ds(q, 16)], localv, mask=m2)
            return q + cnt2

        if first:
            pltpu.make_async_copy(zshared, buf, zsem).wait()

        @pl.loop(0, (q + 15) // 16)
        def _(g):
            localv = ebuf[pl.ds(g * 16, 16)]
            mt = iota16 < jnp.minimum(q - g * 16, 16)
            idxv = localv
            for j in range(L):
                if j:
                    idxv = idxv + 1
                plsc.store_scatter(buf, [idxv], ones16, mask=mt)

        d = store_desc(buf, sbd)
        d.start()
        if not first:
            d.wait()

    pf = (jnp.int32(0), jnp.int32(NCHUNK), jnp.int32(0))
    stage_desc(0, xy0).start()

    @pl.loop(0, NSEG // 2, init_carry=pf)
    def pf0(t, pf):
        s0 = 2 * t
        stage_desc(s0, xy0).wait()
        stage_desc(s0 + 1, xy1).start()
        pf = scan_seg(s0, xy0, pf, 4)
        stage_desc(s0 + 1, xy1).wait()
        pl.when(s0 + 2 < NSEG)(lambda: stage_desc(s0 + 2, xy0).start())
        return scan_seg(s0 + 1, xy1, pf, 4)

    p, first_skip, _ = pf0
    lbuf[pl.ds(p, 16)] = neg16
    nlc = (p + 15) // 16

    @pl.loop(0, SB // 2)
    def _(t):
        build(nlc, 2 * t, rbuf0, True, t > 0)
        build(nlc, 2 * t + 1, rbuf1, True, t > 0)

    store_desc(rbuf0, 0).wait()
    store_desc(rbuf1, 0).wait()

    @pl.loop(1, NB, init_carry=first_skip)
    def final_cur(b, cur):
        pf = (jnp.int32(0), jnp.int32(NCHUNK), cur)

        @pl.loop(0, NSEG, init_carry=pf)
        def pfb(seg, pf):
            def stage():
                d = stage_desc(seg, xy0)
                d.start()
                d.wait()
            pl.when((seg + 1) * CPS > pf[2])(stage)
            return scan_seg(seg, xy0, pf, None)

        p, first_skip, _ = pfb
        lbuf[pl.ds(p, 16)] = neg16
        nlc = (p + 15) // 16

        def rebuild():
            @pl.loop(0, SB)
            def _(sbd):
                build(nlc, sbd, rbuf0, False, None)
        pl.when(p > 0)(rebuild)
        return first_skip


@jax.jit
def kernel(edge_params, velocities):
    ep = edge_params.astype(jnp.int32)
    x0 = jnp.pad(ep[:, 0, 0], (0, NPAD - N)).reshape(NSEG, 1, SEGE)
    y = jnp.pad(ep[:, 0, 1], (0, NPAD - N),
                constant_values=-1).reshape(NSEG, 1, SEGE)
    xy = jnp.concatenate([x0, y], axis=1).reshape(-1)

    mesh = plsc.VectorSubcoreMesh(
        core_axis_name="c", subcore_axis_name="s",
        num_cores=NC, num_subcores=NS)
    out = pl.kernel(
        _sc_body,
        out_type=jax.ShapeDtypeStruct((H * W,), jnp.float32),
        mesh=mesh,
        compiler_params=pltpu.CompilerParams(needs_layout_passes=False),
        scratch_types=[
            pltpu.VMEM((2 * SEGE,), jnp.int32),
            pltpu.VMEM((2 * SEGE,), jnp.int32),
            pltpu.VMEM((CAPL + 16,), jnp.int32),
            pltpu.VMEM((CAPL + 16,), jnp.int32),
            pltpu.VMEM((SB_WORDS,), jnp.float32),
            pltpu.VMEM((SB_WORDS,), jnp.float32),
            pltpu.VMEM((ZVW,), jnp.float32),
            pltpu.VMEM_SHARED((SB_WORDS,), jnp.float32),
            pltpu.SemaphoreType.DMA,
            pltpu.SemaphoreType.DMA,
            pltpu.SemaphoreType.DMA,
        ],
    )(xy)
    return out.reshape(H, W)


# speedup vs baseline: 124.7150x; 1.2273x over previous
import jax
import jax.numpy as jnp
from jax import lax
from jax.experimental import pallas as pl
from jax.experimental.pallas import tpu as pltpu
from jax.experimental.pallas import tpu_sc as plsc

H = 4096
W = 4096
L = 64
N = 100000

NC = 2
NS = 16
NW = NC * NS
ROWS_PER_TILE = H // NW
SB = 16
SB_ROWS = ROWS_PER_TILE // SB
SB_WORDS = SB_ROWS * W

SEGE = 7168
NSEG = 14
NPAD = NSEG * SEGE
CPS = SEGE // 16
NCHUNK = NPAD // 16

CAPL = 12288
NB = 10
ZVW = 4096


def _sc_body(xy_hbm, out_hbm,
             xy0, xy1, lbuf, ebuf, rbuf0, rbuf1, zvm, zshared,
             ssem, gsem, zsem):
    c = lax.axis_index("c")
    s = lax.axis_index("s")
    wid = c * NS + s
    row0 = wid * ROWS_PER_TILE
    base_word = row0 * W
    ones16 = jnp.ones((16,), jnp.float32)
    zeros16 = jnp.zeros((16,), jnp.float32)
    neg16 = jnp.full((16,), -1, jnp.int32)
    iota16 = lax.broadcasted_iota(jnp.int32, (16,), 0)

    @pl.loop(0, ZVW // 16, unroll=8)
    def _(i):
        zvm[pl.ds(i * 16, 16)] = zeros16

    @pl.when(s == 0)
    def _():
        for z in range(SB_WORDS // ZVW):
            pltpu.sync_copy(zvm, zshared.at[pl.ds(z * ZVW, ZVW)])
    plsc.subcore_barrier()

    def store_desc(buf, sbd):
        return pltpu.make_async_copy(
            buf, out_hbm.at[pl.ds(base_word + sbd * SB_WORDS, SB_WORDS)],
            ssem)

    def stage_desc(seg, buf):
        return pltpu.make_async_copy(
            xy_hbm.at[pl.ds(seg * 2 * SEGE, 2 * SEGE)], buf, gsem)

    def scan_seg(segd, buf, pf, unroll):
        lol = 0 if unroll else jnp.clip(pf[2] - segd * CPS, 0, CPS)

        @pl.loop(lol, CPS, init_carry=pf, unroll=unroll)
        def out_pf(lk, pf):
            p, fs, cur = pf
            yv = buf[pl.ds(SEGE + lk * 16, 16)]
            x0v = buf[pl.ds(lk * 16, 16)]
            mv = (yv - row0).astype(jnp.uint32) < ROWS_PER_TILE
            cnt = plsc.all_reduce_population_count(mv)[0]
            plsc.store_compressed(
                lbuf.at[pl.ds(jnp.minimum(p, CAPL), 16)],
                yv * W + x0v, mask=mv)
            fits = (p + cnt) <= CAPL
            p = jnp.where(fits, p + cnt, p)
            fs = jnp.where(
                (~fits) & (fs == NCHUNK), segd * CPS + lk, fs)
            return (p, fs, cur)

        return out_pf

    def build(nlc, sbd, buf, first, wait_prev):
        if wait_prev is not None:
            pl.when(wait_prev)(lambda: store_desc(buf, 0).wait())
        base = base_word + sbd * SB_WORDS
        if first:
            pltpu.make_async_copy(zshared, buf, zsem).start()
        else:
            pltpu.sync_copy(out_hbm.at[pl.ds(base, SB_WORDS)], buf)

        @pl.loop(0, nlc, init_carry=jnp.int32(0))
        def q(lc, q):
            localv = lbuf[pl.ds(lc * 16, 16)] - base
            m2 = localv.astype(jnp.uint32) < SB_WORDS
            cnt2 = plsc.all_reduce_population_count(m2)[0]

            @pl.when(cnt2 > 0)
            def _():
                plsc.store_compressed(ebuf.at[pl.ds(q, 16)], localv, mask=m2)
            return q + cnt2

        if first:
            pltpu.make_async_copy(zshared, buf, zsem).wait()

        @pl.loop(0, (q + 15) // 16)
        def _(g):
            localv = ebuf[pl.ds(g * 16, 16)]
            mt = iota16 < jnp.minimum(q - g * 16, 16)
            idxv = localv
            for j in range(L):
                if j:
                    idxv = idxv + 1
                plsc.store_scatter(buf, [idxv], ones16, mask=mt)

        d = store_desc(buf, sbd)
        d.start()
        if not first:
            d.wait()

    pf = (jnp.int32(0), jnp.int32(NCHUNK), jnp.int32(0))
    stage_desc(0, xy0).start()

    @pl.loop(0, NSEG // 2, init_carry=pf)
    def pf0(t, pf):
        s0 = 2 * t
        stage_desc(s0, xy0).wait()
        stage_desc(s0 + 1, xy1).start()
        pf = scan_seg(s0, xy0, pf, 8)
        stage_desc(s0 + 1, xy1).wait()
        pl.when(s0 + 2 < NSEG)(lambda: stage_desc(s0 + 2, xy0).start())
        return scan_seg(s0 + 1, xy1, pf, 8)

    p, first_skip, _ = pf0
    lbuf[pl.ds(p, 16)] = neg16
    nlc = (p + 15) // 16

    @pl.loop(0, SB // 2)
    def _(t):
        build(nlc, 2 * t, rbuf0, True, t > 0)
        build(nlc, 2 * t + 1, rbuf1, True, t > 0)

    store_desc(rbuf0, 0).wait()
    store_desc(rbuf1, 0).wait()

    @pl.loop(1, NB, init_carry=first_skip)
    def final_cur(b, cur):
        pf = (jnp.int32(0), jnp.int32(NCHUNK), cur)

        @pl.loop(0, NSEG, init_carry=pf)
        def pfb(seg, pf):
            def stage():
                d = stage_desc(seg, xy0)
                d.start()
                d.wait()
            pl.when((seg + 1) * CPS > pf[2])(stage)
            return scan_seg(seg, xy0, pf, None)

        p, first_skip, _ = pfb
        lbuf[pl.ds(p, 16)] = neg16
        nlc = (p + 15) // 16

        def rebuild():
            @pl.loop(0, SB)
            def _(sbd):
                build(nlc, sbd, rbuf0, False, None)
        pl.when(p > 0)(rebuild)
        return first_skip


@jax.jit
def kernel(edge_params, velocities):
    ep = edge_params.astype(jnp.int32)
    x0 = jnp.pad(ep[:, 0, 0], (0, NPAD - N)).reshape(NSEG, 1, SEGE)
    y = jnp.pad(ep[:, 0, 1], (0, NPAD - N),
                constant_values=-1).reshape(NSEG, 1, SEGE)
    xy = jnp.concatenate([x0, y], axis=1).reshape(-1)

    mesh = plsc.VectorSubcoreMesh(
        core_axis_name="c", subcore_axis_name="s",
        num_cores=NC, num_subcores=NS)
    out = pl.kernel(
        _sc_body,
        out_type=jax.ShapeDtypeStruct((H * W,), jnp.float32),
        mesh=mesh,
        compiler_params=pltpu.CompilerParams(needs_layout_passes=False),
        scratch_types=[
            pltpu.VMEM((2 * SEGE,), jnp.int32),
            pltpu.VMEM((2 * SEGE,), jnp.int32),
            pltpu.VMEM((CAPL + 16,), jnp.int32),
            pltpu.VMEM((CAPL + 16,), jnp.int32),
            pltpu.VMEM((SB_WORDS,), jnp.float32),
            pltpu.VMEM((SB_WORDS,), jnp.float32),
            pltpu.VMEM((ZVW,), jnp.float32),
            pltpu.VMEM_SHARED((SB_WORDS,), jnp.float32),
            pltpu.SemaphoreType.DMA,
            pltpu.SemaphoreType.DMA,
            pltpu.SemaphoreType.DMA,
        ],
    )(xy)
    return out.reshape(H, W)


# speedup vs baseline: 126.2904x; 1.0126x over previous
import jax
import jax.numpy as jnp
from jax import lax
from jax.experimental import pallas as pl
from jax.experimental.pallas import tpu as pltpu
from jax.experimental.pallas import tpu_sc as plsc

H = 4096
W = 4096
L = 64
N = 100000

NC = 2
NS = 16
NW = NC * NS
ROWS_PER_TILE = H // NW
SB = 16
SB_ROWS = ROWS_PER_TILE // SB
SB_WORDS = SB_ROWS * W

SEGE = 7168
NSEG = 14
NPAD = NSEG * SEGE
CPS = SEGE // 16
NCHUNK = NPAD // 16

CAPL = 12288
NB = NSEG + 1
ZVW = 4096


def _sc_body(xy_hbm, out_hbm,
             xy0, xy1, lbuf, ebuf, rbuf0, rbuf1, zvm, zshared,
             ssem, gsem, zsem):
    c = lax.axis_index("c")
    s = lax.axis_index("s")
    wid = c * NS + s
    row0 = wid * ROWS_PER_TILE
    base_word = row0 * W
    ones16 = jnp.ones((16,), jnp.float32)
    zeros16 = jnp.zeros((16,), jnp.float32)
    neg16 = jnp.full((16,), -1, jnp.int32)
    iota16 = lax.broadcasted_iota(jnp.int32, (16,), 0)

    @pl.loop(0, ZVW // 16, unroll=8)
    def _(i):
        zvm[pl.ds(i * 16, 16)] = zeros16

    @pl.when(s == 0)
    def _():
        for z in range(SB_WORDS // ZVW):
            pltpu.sync_copy(zvm, zshared.at[pl.ds(z * ZVW, ZVW)])
    plsc.subcore_barrier()

    def store_desc(buf, sbd):
        return pltpu.make_async_copy(
            buf, out_hbm.at[pl.ds(base_word + sbd * SB_WORDS, SB_WORDS)],
            ssem)

    def stage_desc(seg, buf):
        return pltpu.make_async_copy(
            xy_hbm.at[pl.ds(seg * 2 * SEGE, 2 * SEGE)], buf, gsem)

    def scan_seg(segd, buf, pf, unroll):
        lol = 0 if unroll else jnp.clip(pf[2] - segd * CPS, 0, CPS)
        pb0 = (pf[0], (pf[0] < 0))

        @pl.loop(lol, CPS, init_carry=pb0, unroll=unroll)
        def out_pb(lk, pb):
            p, bad = pb
            yv = buf[pl.ds(SEGE + lk * 16, 16)]
            x0v = buf[pl.ds(lk * 16, 16)]
            mv = (yv - row0).astype(jnp.uint32) < ROWS_PER_TILE
            cnt = plsc.all_reduce_population_count(mv)[0]
            plsc.store_compressed(
                lbuf.at[pl.ds(p, 16)], yv * W + x0v, mask=mv)
            fits = (p + cnt) <= CAPL
            p = jnp.where(fits, p + cnt, p)
            return (p, bad | ~fits)

        p, bad = out_pb
        fs = jnp.where(bad & (pf[1] == NCHUNK), segd * CPS, pf[1])
        return (p, fs, pf[2])

    def build(nlc, sbd, buf, first, wait_prev):
        if wait_prev is not None:
            pl.when(wait_prev)(lambda: store_desc(buf, 0).wait())
        base = base_word + sbd * SB_WORDS
        if first:
            pltpu.make_async_copy(zshared, buf, zsem).start()
        else:
            pltpu.sync_copy(out_hbm.at[pl.ds(base, SB_WORDS)], buf)

        @pl.loop(0, nlc, init_carry=jnp.int32(0))
        def q(lc, q):
            localv = lbuf[pl.ds(lc * 16, 16)] - base
            m2 = localv.astype(jnp.uint32) < SB_WORDS
            cnt2 = plsc.all_reduce_population_count(m2)[0]

            @pl.when(cnt2 > 0)
            def _():
                plsc.store_compressed(ebuf.at[pl.ds(q, 16)], localv, mask=m2)
            return q + cnt2

        if first:
            pltpu.make_async_copy(zshared, buf, zsem).wait()

        @pl.loop(0, (q + 15) // 16)
        def _(g):
            localv = ebuf[pl.ds(g * 16, 16)]
            mt = iota16 < jnp.minimum(q - g * 16, 16)
            idxv = localv
            for j in range(L):
                if j:
                    idxv = idxv + 1
                plsc.store_scatter(buf, [idxv], ones16, mask=mt)

        d = store_desc(buf, sbd)
        d.start()
        if not first:
            d.wait()

    pf = (jnp.int32(0), jnp.int32(NCHUNK), jnp.int32(0))
    stage_desc(0, xy0).start()

    @pl.loop(0, NSEG // 2, init_carry=pf)
    def pf0(t, pf):
        s0 = 2 * t
        stage_desc(s0, xy0).wait()
        stage_desc(s0 + 1, xy1).start()
        pf = scan_seg(s0, xy0, pf, 16)
        stage_desc(s0 + 1, xy1).wait()
        pl.when(s0 + 2 < NSEG)(lambda: stage_desc(s0 + 2, xy0).start())
        return scan_seg(s0 + 1, xy1, pf, 16)

    p, first_skip, _ = pf0
    lbuf[pl.ds(p, 16)] = neg16
    nlc = (p + 15) // 16

    @pl.loop(0, SB // 2)
    def _(t):
        build(nlc, 2 * t, rbuf0, True, t > 0)
        build(nlc, 2 * t + 1, rbuf1, True, t > 0)

    store_desc(rbuf0, 0).wait()
    store_desc(rbuf1, 0).wait()

    @pl.loop(1, NB, init_carry=first_skip)
    def final_cur(b, cur):
        pf = (jnp.int32(0), jnp.int32(NCHUNK), cur)

        @pl.loop(0, NSEG, init_carry=pf)
        def pfb(seg, pf):
            def stage():
                d = stage_desc(seg, xy0)
                d.start()
                d.wait()
            pl.when((seg + 1) * CPS > pf[2])(stage)
            return scan_seg(seg, xy0, pf, None)

        p, first_skip, _ = pfb
        lbuf[pl.ds(p, 16)] = neg16
        nlc = (p + 15) // 16

        def rebuild():
            @pl.loop(0, SB)
            def _(sbd):
                build(nlc, sbd, rbuf0, False, None)
        pl.when(p > 0)(rebuild)
        return first_skip


@jax.jit
def kernel(edge_params, velocities):
    ep = edge_params.astype(jnp.int32)
    x0 = jnp.pad(ep[:, 0, 0], (0, NPAD - N)).reshape(NSEG, 1, SEGE)
    y = jnp.pad(ep[:, 0, 1], (0, NPAD - N),
                constant_values=-1).reshape(NSEG, 1, SEGE)
    xy = jnp.concatenate([x0, y], axis=1).reshape(-1)

    mesh = plsc.VectorSubcoreMesh(
        core_axis_name="c", subcore_axis_name="s",
        num_cores=NC, num_subcores=NS)
    out = pl.kernel(
        _sc_body,
        out_type=jax.ShapeDtypeStruct((H * W,), jnp.float32),
        mesh=mesh,
        compiler_params=pltpu.CompilerParams(needs_layout_passes=False),
        scratch_types=[
            pltpu.VMEM((2 * SEGE,), jnp.int32),
            pltpu.VMEM((2 * SEGE,), jnp.int32),
            pltpu.VMEM((CAPL + 16,), jnp.int32),
            pltpu.VMEM((CAPL + 16,), jnp.int32),
            pltpu.VMEM((SB_WORDS,), jnp.float32),
            pltpu.VMEM((SB_WORDS,), jnp.float32),
            pltpu.VMEM((ZVW,), jnp.float32),
            pltpu.VMEM_SHARED((SB_WORDS,), jnp.float32),
            pltpu.SemaphoreType.DMA,
            pltpu.SemaphoreType.DMA,
            pltpu.SemaphoreType.DMA,
        ],
    )(xy)
    return out.reshape(H, W)
